# baseline (speedup 1.0000x reference)
import jax
import jax.numpy as jnp
from jax import lax
from jax.experimental import pallas as pl
from jax.experimental.pallas import tpu as pltpu
from jax.experimental.pallas import tpu_sc as plsc

_HIDDEN = 64
_BATCH = 16384
_RMIN, _RMAX = 1.0, 5.0

_NC = 2
_NS = 16
_L = 16
_NW = _NC * _NS
_BPW = _BATCH // _NW
_CH = 128
_NCHUNK = _BPW // _CH
_GPC = _CH // _L


def _mf_body(user_hbm, item_hbm, rating_hbm, cat_hbm,
             tr_hbm, part_hbm,
             uidx_v, iidx_v, ue_v, ie_v, rate_v, tr_v, part_v,
             sem0, sem1):
    wid = lax.axis_index("s") * _NC + lax.axis_index("c")
    base = wid * _BPW
    sems = (sem0, sem1)

    for c in range(_NCHUNK):
        pltpu.sync_copy(user_hbm.at[pl.ds(base + c * _CH, _CH)],
                        uidx_v.at[c])
        pltpu.sync_copy(item_hbm.at[pl.ds(base + c * _CH, _CH)],
                        iidx_v.at[c])
    pltpu.sync_copy(rating_hbm.at[pl.ds(base, _BPW)], rate_v)

    def issue(c, buf):
        sem = sems[c & 1]
        pltpu.async_copy(cat_hbm.at[uidx_v.at[c]],
                         ue_v.at[pl.ds(buf * _CH, _CH)], sem)
        pltpu.async_copy(cat_hbm.at[iidx_v.at[c]],
                         ie_v.at[pl.ds(buf * _CH, _CH)], sem)

    def drain(c, buf):
        sem = sems[c & 1]
        pltpu.make_async_copy(cat_hbm.at[pl.ds(0, _CH)],
                              ue_v.at[pl.ds(buf * _CH, _CH)], sem).wait()
        pltpu.make_async_copy(cat_hbm.at[pl.ds(0, _CH)],
                              ie_v.at[pl.ds(buf * _CH, _CH)], sem).wait()

    lanes = lax.iota(jnp.int32, _L)
    inv_h = jnp.float32(1.0 / _HIDDEN)
    inv_span = jnp.float32(1.0 / (_RMAX - _RMIN))

    def compute(c, buf, acc):

        def group_body(g, acc):
            rows = buf * _CH + g * _L + lanes

            def j_body(j, carry):
                su, sv, suu, svv, suv = carry
                for jj in range(8):
                    cols = (lanes + (j * 8 + jj)) & (_HIDDEN - 1)
                    u = plsc.load_gather(ue_v, [rows, cols])
                    v = plsc.load_gather(ie_v, [rows, cols + _HIDDEN])
                    su = su + u
                    sv = sv + v
                    suu = suu + u * u
                    svv = svv + v * v
                    suv = suv + u * v
                return (su, sv, suu, svv, suv)

            z = jnp.zeros((_L,), jnp.float32)
            su, sv, suu, svv, suv = lax.fori_loop(
                0, _HIDDEN // 8, j_body, (z, z, z, z, z))

            dotc = suv - su * sv * inv_h
            nu = jnp.maximum(suu - su * su * inv_h, jnp.float32(1e-24))
            nv = jnp.maximum(svv - sv * sv * inv_h, jnp.float32(1e-24))
            d = jnp.maximum(nu * nv, jnp.float32(1e-30))
            yi = jnp.int32(0x5F3759DF) - (plsc.bitcast(d, jnp.int32) >> 1)
            y = plsc.bitcast(yi, jnp.float32)
            for _ in range(3):
                y = y * (jnp.float32(1.5) - jnp.float32(0.5) * d * y * y)
            mf = dotc * y

            off = c * _CH + g * _L
            tr_v[pl.ds(off, _L)] = mf * jnp.float32(_RMAX - _RMIN) \
                + jnp.float32(_RMIN)
            r = (rate_v[pl.ds(off, _L)] - jnp.float32(_RMIN)) * inv_span
            e = mf - r
            return acc + e * e

        return lax.fori_loop(0, _GPC, group_body, acc)

    issue(0, 0)
    acc = jnp.zeros((_L,), jnp.float32)
    for c in range(_NCHUNK):
        buf = c & 1
        if c + 1 < _NCHUNK:
            issue(c + 1, 1 - buf)
        drain(c, buf)
        acc = compute(c, buf, acc)

    part_v[...] = acc
    pltpu.sync_copy(tr_v, tr_hbm.at[pl.ds(base, _BPW)])
    pltpu.sync_copy(part_v, part_hbm.at[pl.ds(wid * _L, _L)])


_BK = 16384
_NV = 100000
_NBLK = (_NV + _BK - 1) // _BK
_NPAD = _NBLK * _BK


def _transpose_cat_body(ut_ref, it_ref, out_ref):
    out_ref[...] = jnp.concatenate(
        [ut_ref[...].T, it_ref[...].T], axis=1)


@jax.jit
def kernel(user, item, rating, user_weight, item_weight):
    cat = pl.pallas_call(
        _transpose_cat_body,
        grid=(_NBLK,),
        in_specs=[
            pl.BlockSpec((_HIDDEN, _BK), lambda k: (0, k)),
            pl.BlockSpec((_HIDDEN, _BK), lambda k: (0, k)),
        ],
        out_specs=pl.BlockSpec((_BK, 128), lambda k: (k, 0)),
        out_shape=jax.ShapeDtypeStruct((_NPAD, 128), jnp.float32),
    )(user_weight.T, item_weight.T)
    mesh = plsc.VectorSubcoreMesh(core_axis_name="c", subcore_axis_name="s")
    tr, part = pl.kernel(
        _mf_body,
        out_type=[
            jax.ShapeDtypeStruct((_BATCH,), jnp.float32),
            jax.ShapeDtypeStruct((_NW * _L,), jnp.float32),
        ],
        mesh=mesh,
        compiler_params=pltpu.CompilerParams(needs_layout_passes=False),
        scratch_types=[
            pltpu.VMEM((_NCHUNK, _CH), jnp.int32),
            pltpu.VMEM((_NCHUNK, _CH), jnp.int32),
            pltpu.VMEM((2 * _CH, 128), jnp.float32),
            pltpu.VMEM((2 * _CH, 128), jnp.float32),
            pltpu.VMEM((_BPW,), jnp.float32),
            pltpu.VMEM((_BPW,), jnp.float32),
            pltpu.VMEM((_L,), jnp.float32),
            pltpu.SemaphoreType.DMA,
            pltpu.SemaphoreType.DMA,
        ],
    )(user, item, rating, cat)
    loss = jnp.sum(part) * jnp.float32(1.0 / _BATCH)
    return (loss, tr)


# speedup vs baseline: 1.0372x; 1.0372x over previous
import jax
import jax.numpy as jnp
from jax import lax
from jax.experimental import pallas as pl
from jax.experimental.pallas import tpu as pltpu
from jax.experimental.pallas import tpu_sc as plsc

_HIDDEN = 64
_BATCH = 16384
_RMIN, _RMAX = 1.0, 5.0

_NC = 2
_NS = 16
_L = 16
_NW = _NC * _NS
_BPW = _BATCH // _NW
_CH = 128
_NCHUNK = _BPW // _CH
_GPC = _CH // _L


def _mf_body(user_hbm, item_hbm, rating_hbm, cat_hbm,
             tr_hbm, part_hbm,
             uidx_v, iidx_v, ue_v, ie_v, rate_v, tr_v, part_v,
             sem0, sem1):
    wid = lax.axis_index("s") * _NC + lax.axis_index("c")
    base = wid * _BPW
    sems = (sem0, sem1)

    for c in range(_NCHUNK):
        pltpu.sync_copy(user_hbm.at[pl.ds(base + c * _CH, _CH)],
                        uidx_v.at[c])
        pltpu.sync_copy(item_hbm.at[pl.ds(base + c * _CH, _CH)],
                        iidx_v.at[c])
    pltpu.sync_copy(rating_hbm.at[pl.ds(base, _BPW)], rate_v)

    def issue(c, buf):
        sem = sems[c & 1]
        pltpu.async_copy(cat_hbm.at[uidx_v.at[c]],
                         ue_v.at[pl.ds(buf * _CH, _CH)], sem)
        pltpu.async_copy(cat_hbm.at[iidx_v.at[c]],
                         ie_v.at[pl.ds(buf * _CH, _CH)], sem)

    def drain(c, buf):
        sem = sems[c & 1]
        pltpu.make_async_copy(cat_hbm.at[pl.ds(0, _CH)],
                              ue_v.at[pl.ds(buf * _CH, _CH)], sem).wait()
        pltpu.make_async_copy(cat_hbm.at[pl.ds(0, _CH)],
                              ie_v.at[pl.ds(buf * _CH, _CH)], sem).wait()

    lanes = lax.iota(jnp.int32, _L)
    inv_h = jnp.float32(1.0 / _HIDDEN)
    inv_span = jnp.float32(1.0 / (_RMAX - _RMIN))

    def compute(c, buf, acc):

        def group_body(g, acc):
            rows = buf * _CH + g * _L + lanes

            def j_body(j, carry):
                su, sv, suu, svv, suv = carry
                for jj in range(8):
                    cols = (lanes + (j * 8 + jj)) & (_HIDDEN - 1)
                    u = plsc.load_gather(ue_v, [rows, cols])
                    v = plsc.load_gather(ie_v, [rows, cols + _HIDDEN])
                    su = su + u
                    sv = sv + v
                    suu = suu + u * u
                    svv = svv + v * v
                    suv = suv + u * v
                return (su, sv, suu, svv, suv)

            z = jnp.zeros((_L,), jnp.float32)
            su, sv, suu, svv, suv = lax.fori_loop(
                0, _HIDDEN // 8, j_body, (z, z, z, z, z))

            dotc = suv - su * sv * inv_h
            nu = jnp.maximum(suu - su * su * inv_h, jnp.float32(1e-24))
            nv = jnp.maximum(svv - sv * sv * inv_h, jnp.float32(1e-24))
            d = jnp.maximum(nu * nv, jnp.float32(1e-30))
            yi = jnp.int32(0x5F3759DF) - (plsc.bitcast(d, jnp.int32) >> 1)
            y = plsc.bitcast(yi, jnp.float32)
            for _ in range(3):
                y = y * (jnp.float32(1.5) - jnp.float32(0.5) * d * y * y)
            mf = dotc * y

            off = c * _CH + g * _L
            tr_v[pl.ds(off, _L)] = mf * jnp.float32(_RMAX - _RMIN) \
                + jnp.float32(_RMIN)
            r = (rate_v[pl.ds(off, _L)] - jnp.float32(_RMIN)) * inv_span
            e = mf - r
            return acc + e * e

        return lax.fori_loop(0, _GPC, group_body, acc)

    issue(0, 0)
    acc = jnp.zeros((_L,), jnp.float32)
    for c in range(_NCHUNK):
        buf = c & 1
        if c + 1 < _NCHUNK:
            issue(c + 1, 1 - buf)
        drain(c, buf)
        acc = compute(c, buf, acc)

    part_v[...] = acc
    pltpu.sync_copy(tr_v, tr_hbm.at[pl.ds(base, _BPW)])
    pltpu.sync_copy(part_v, part_hbm.at[pl.ds(wid * _L, _L)])


_BK = 8192
_NV = 100000
_NBLK = (_NV + _BK - 1) // _BK
_NPAD = _NBLK * _BK


def _transpose_cat_body(ut_ref, it_ref, out_ref):
    out_ref[...] = jnp.concatenate(
        [ut_ref[...].T, it_ref[...].T], axis=1)


@jax.jit
def kernel(user, item, rating, user_weight, item_weight):
    cat = pl.pallas_call(
        _transpose_cat_body,
        grid=(_NBLK,),
        in_specs=[
            pl.BlockSpec((_HIDDEN, _BK), lambda k: (0, k)),
            pl.BlockSpec((_HIDDEN, _BK), lambda k: (0, k)),
        ],
        out_specs=pl.BlockSpec((_BK, 128), lambda k: (k, 0)),
        out_shape=jax.ShapeDtypeStruct((_NPAD, 128), jnp.float32),
    )(user_weight.T, item_weight.T)
    mesh = plsc.VectorSubcoreMesh(core_axis_name="c", subcore_axis_name="s")
    tr, part = pl.kernel(
        _mf_body,
        out_type=[
            jax.ShapeDtypeStruct((_BATCH,), jnp.float32),
            jax.ShapeDtypeStruct((_NW * _L,), jnp.float32),
        ],
        mesh=mesh,
        compiler_params=pltpu.CompilerParams(needs_layout_passes=False),
        scratch_types=[
            pltpu.VMEM((_NCHUNK, _CH), jnp.int32),
            pltpu.VMEM((_NCHUNK, _CH), jnp.int32),
            pltpu.VMEM((2 * _CH, 128), jnp.float32),
            pltpu.VMEM((2 * _CH, 128), jnp.float32),
            pltpu.VMEM((_BPW,), jnp.float32),
            pltpu.VMEM((_BPW,), jnp.float32),
            pltpu.VMEM((_L,), jnp.float32),
            pltpu.SemaphoreType.DMA,
            pltpu.SemaphoreType.DMA,
        ],
    )(user, item, rating, cat)
    loss = jnp.sum(part) * jnp.float32(1.0 / _BATCH)
    return (loss, tr)
